# baseline (speedup 1.0000x reference)
import functools

import jax
import jax.numpy as jnp
from jax import lax
from jax.experimental import pallas as pl
from jax.experimental.pallas import tpu as pltpu
from jax.experimental.pallas import tpu_sc as plsc

D_MODEL = 128


def _build_sc_gather(V, E, B, S):
    info = plsc.get_sparse_core_info()
    NC, NS = info.num_cores, info.num_subcores
    NW = NC * NS
    n_groups = NW // S
    assert B % (8 * n_groups) == 0
    b_per_g = B // n_groups
    mesh = plsc.VectorSubcoreMesh(core_axis_name="c", subcore_axis_name="s")

    @functools.partial(
        pl.kernel,
        out_type=jax.ShapeDtypeStruct((B, S * E), jnp.float32),
        mesh=mesh,
        compiler_params=pltpu.CompilerParams(
            use_tc_tiling_on_sc=False, needs_layout_passes=False
        ),
        scratch_types=[
            pltpu.VMEM((b_per_g, S), jnp.int32),
            pltpu.VMEM((b_per_g,), jnp.int32),
            pltpu.VMEM((b_per_g, E), jnp.float32),
            pltpu.SemaphoreType.DMA,
        ],
    )
    def gather_kernel(table_hbm, idx_hbm, out_hbm, idx2d, idx_v, rows_v, sem):
        wid = lax.axis_index("s") * NC + lax.axis_index("c")
        s = wid % S
        base = (wid // S) * b_per_g
        pltpu.sync_copy(idx_hbm.at[pl.ds(base, b_per_g)], idx2d)
        lanes = lax.iota(jnp.int32, 16)
        col = jnp.full((16,), s, jnp.int32)

        def repack(i, _):
            row = i * 16 + lanes
            idx_v[pl.ds(i * 16, 16)] = plsc.load_gather(idx2d, [row, col])
            return _

        lax.fori_loop(0, b_per_g // 16, repack, None, unroll=4)
        pltpu.async_copy(table_hbm.at[idx_v], rows_v, sem).wait()
        pltpu.sync_copy(
            rows_v, out_hbm.at[pl.ds(base, b_per_g), pl.ds(s * E, E)]
        )

    return gather_kernel


def _mm_body(e_ref, w_ref, b_ref, o_ref):
    o_ref[...] = lax.dot_general(
        e_ref[...], w_ref[...],
        dimension_numbers=(((1,), (1,)), ((), ())),
        preferred_element_type=jnp.float32,
    ) + b_ref[...]


def _tc_project(e, W, b2d, block_m):
    B = e.shape[0]
    return pl.pallas_call(
        _mm_body,
        out_shape=jax.ShapeDtypeStruct((B, D_MODEL), jnp.float32),
        grid=(B // block_m,),
        in_specs=[
            pl.BlockSpec((block_m, D_MODEL), lambda i: (i, 0)),
            pl.BlockSpec((D_MODEL, D_MODEL), lambda i: (0, 0)),
            pl.BlockSpec((1, D_MODEL), lambda i: (0, 0)),
        ],
        out_specs=pl.BlockSpec((block_m, D_MODEL), lambda i: (i, 0)),
    )(e, W, b2d)


def kernel(ctrl_tokens, embed_table, W, b):
    B, S = ctrl_tokens.shape
    V, E = embed_table.shape
    idx = ctrl_tokens.astype(jnp.int32)
    e = _build_sc_gather(V, E, B, S)(embed_table, idx)
    out = _tc_project(e, W, b.reshape(1, D_MODEL), 2048)
    return out[..., None]


# speedup vs baseline: 1.1517x; 1.1517x over previous
import functools

import jax
import jax.numpy as jnp
from jax import lax
from jax.experimental import pallas as pl
from jax.experimental.pallas import tpu as pltpu
from jax.experimental.pallas import tpu_sc as plsc

D_MODEL = 128


def _build_sc_gather(V, E, B, S):
    info = plsc.get_sparse_core_info()
    NC, NS = info.num_cores, info.num_subcores
    NW = NC * NS
    n_groups = NW // S
    assert B % (8 * n_groups) == 0
    b_per_g = B // n_groups
    mesh = plsc.VectorSubcoreMesh(core_axis_name="c", subcore_axis_name="s")

    @functools.partial(
        pl.kernel,
        out_type=jax.ShapeDtypeStruct((B, S * E), jnp.float32),
        mesh=mesh,
        compiler_params=pltpu.CompilerParams(
            use_tc_tiling_on_sc=False, needs_layout_passes=False
        ),
        scratch_types=[
            pltpu.VMEM((b_per_g,), jnp.int32),
            pltpu.VMEM((b_per_g, E), jnp.float32),
            pltpu.SemaphoreType.DMA,
        ],
    )
    def gather_kernel(
        table_hbm, i0_hbm, i1_hbm, i2_hbm, i3_hbm, out_hbm, idx_v, rows_v, sem
    ):
        wid = lax.axis_index("s") * NC + lax.axis_index("c")
        s = wid % S
        base = (wid // S) * b_per_g
        idx_refs = [i0_hbm, i1_hbm, i2_hbm, i3_hbm]
        for si in range(S):
            @pl.when(s == si)
            def _():
                pltpu.sync_copy(idx_refs[si].at[pl.ds(base, b_per_g)], idx_v)
        pltpu.async_copy(table_hbm.at[idx_v], rows_v, sem).wait()
        pltpu.sync_copy(
            rows_v, out_hbm.at[pl.ds(base, b_per_g), pl.ds(s * E, E)]
        )

    return gather_kernel


def _mm_body(e_ref, w_ref, b_ref, o_ref):
    o_ref[...] = lax.dot_general(
        e_ref[...], w_ref[...],
        dimension_numbers=(((1,), (1,)), ((), ())),
        preferred_element_type=jnp.float32,
    ) + b_ref[...]


def _tc_project(e, W, b2d, block_m):
    B = e.shape[0]
    return pl.pallas_call(
        _mm_body,
        out_shape=jax.ShapeDtypeStruct((B, D_MODEL), jnp.float32),
        grid=(B // block_m,),
        in_specs=[
            pl.BlockSpec((block_m, D_MODEL), lambda i: (i, 0)),
            pl.BlockSpec((D_MODEL, D_MODEL), lambda i: (0, 0)),
            pl.BlockSpec((1, D_MODEL), lambda i: (0, 0)),
        ],
        out_specs=pl.BlockSpec((block_m, D_MODEL), lambda i: (i, 0)),
    )(e, W, b2d)


def kernel(ctrl_tokens, embed_table, W, b):
    B, S = ctrl_tokens.shape
    V, E = embed_table.shape
    idx = ctrl_tokens.astype(jnp.int32)
    cols = [idx[:, s] for s in range(S)]
    e = _build_sc_gather(V, E, B, S)(embed_table, *cols)
    out = _tc_project(e, W, b.reshape(1, D_MODEL), 2048)
    return out[..., None]
